# baseline (speedup 1.0000x reference)
import functools

import jax
import jax.numpy as jnp
from jax import lax
from jax.experimental import pallas as pl
from jax.experimental.pallas import tpu as pltpu
from jax.experimental.pallas import tpu_sc as plsc

_R, _P, _S, _E = 64, 16, 32, 17
_D = 128
_KP = 144
_OFF_P, _OFF_S, _OFF_E = _R, _R + _P, _R + _P + _S
_LPAD = 56


def _premul_body(tabs_ref, w_ref, out_ref):
    out_ref[...] = jnp.dot(tabs_ref[...], w_ref[...],
                           preferred_element_type=jnp.float32)


def _pack_idx_body(ri_ref, pi_ref, si_ref, ei_ref, out_ref, *, l):
    tb = ri_ref.shape[0]
    pad = jnp.zeros((tb, _D - l), jnp.int32)
    out_ref[:, 0, :] = jnp.concatenate([ri_ref[...], pad], axis=1)
    out_ref[:, 1, :] = jnp.concatenate([pi_ref[...] + _OFF_P, pad], axis=1)
    out_ref[:, 2, :] = jnp.concatenate([si_ref[...] + _OFF_S, pad], axis=1)
    out_ref[:, 3, :] = jnp.concatenate([ei_ref[...] + _OFF_E, pad], axis=1)


def _sc_gather_body(wc_hbm, idx_hbm, y_hbm, idx_s, acc_s, sem_i, sem_g,
                    sem_o, *, rows_per_w, l):
    w = lax.axis_index("s") * 2 + lax.axis_index("c")
    base = w * rows_per_w

    def two_rows(i, carry):
        for s in (0, 1):
            k = i * 2 + s
            b = base + k
            pltpu.async_copy(idx_hbm.at[b], idx_s.at[s], sem_i).wait()
            @pl.when(i >= 1)
            def _():
                pltpu.make_async_copy(
                    y_hbm.at[pl.ds(0, _LPAD), :], acc_s.at[s], sem_o.at[s]).wait()
            pltpu.async_copy(wc_hbm.at[idx_s.at[s, 0, pl.ds(0, _LPAD)]],
                             acc_s.at[s], sem_g).wait()
            c1 = pltpu.async_copy(wc_hbm.at[idx_s.at[s, 1, pl.ds(0, _LPAD)]],
                                  acc_s.at[s], sem_g, add=True)
            c2 = pltpu.async_copy(wc_hbm.at[idx_s.at[s, 2, pl.ds(0, _LPAD)]],
                                  acc_s.at[s], sem_g, add=True)
            c3 = pltpu.async_copy(wc_hbm.at[idx_s.at[s, 3, pl.ds(0, _LPAD)]],
                                  acc_s.at[s], sem_g, add=True)
            c1.wait()
            c2.wait()
            c3.wait()
            pltpu.async_copy(acc_s.at[s], y_hbm.at[pl.ds(b * _LPAD, _LPAD), :],
                             sem_o.at[s])
        return carry

    lax.fori_loop(0, rows_per_w // 2, two_rows, 0)
    for s in (0, 1):
        pltpu.make_async_copy(y_hbm.at[pl.ds(0, _LPAD), :], acc_s.at[s],
                              sem_o.at[s]).wait()


def _ln_body(y_ref, b_ref, g_ref, bb_ref, out_ref, *, tb, l):
    x = y_ref[...] + b_ref[0, :][None, :]
    mu = jnp.mean(x, axis=1, keepdims=True)
    xc = x - mu
    var = jnp.mean(xc * xc, axis=1, keepdims=True)
    y = xc * lax.rsqrt(var + 1e-5) * g_ref[0, :][None, :] \
        + bb_ref[0, :][None, :]
    out_ref[...] = y.reshape(tb, _LPAD, _D)[:, :l, :]


def kernel(root_indices, prefix_indices, suffix_indices, ending_indices,
           root_table, prefix_table, suffix_table, ending_table,
           proj_w, proj_b, ln_gamma, ln_beta):
    b, l = root_indices.shape
    tb = 512
    g = b // tb

    tabs = jnp.concatenate([
        root_table, prefix_table, suffix_table, ending_table,
        jnp.zeros((_KP - _OFF_E - _E, _D), jnp.float32),
    ], axis=0)

    wc = pl.pallas_call(
        _premul_body,
        out_shape=jax.ShapeDtypeStruct((_KP, _D), jnp.float32),
    )(tabs, proj_w)

    idx_spec = pl.BlockSpec((tb, l), lambda i: (i, 0))
    idxpack = pl.pallas_call(
        functools.partial(_pack_idx_body, l=l),
        grid=(g,),
        in_specs=[idx_spec, idx_spec, idx_spec, idx_spec],
        out_specs=pl.BlockSpec((tb, 8, _D), lambda i: (i, 0, 0)),
        out_shape=jax.ShapeDtypeStruct((b, 8, _D), jnp.int32),
    )(root_indices, prefix_indices, suffix_indices, ending_indices)

    rows_per_w = b // 32
    mesh = plsc.VectorSubcoreMesh(core_axis_name="c", subcore_axis_name="s")
    y2 = pl.kernel(
        functools.partial(_sc_gather_body, rows_per_w=rows_per_w, l=l),
        out_type=jax.ShapeDtypeStruct((b * _LPAD, _D), jnp.float32),
        mesh=mesh,
        scratch_types=[
            pltpu.VMEM((2, 8, _D), jnp.int32),
            pltpu.VMEM((2, _LPAD, _D), jnp.float32),
            pltpu.SemaphoreType.DMA,
            pltpu.SemaphoreType.DMA,
            pltpu.SemaphoreType.DMA((2,)),
        ],
    )(wc, idxpack)

    tb2 = 256
    g2 = b // tb2
    vec_spec = pl.BlockSpec((1, _D), lambda i: (0, 0))
    out = pl.pallas_call(
        functools.partial(_ln_body, tb=tb2, l=l),
        grid=(g2,),
        in_specs=[
            pl.BlockSpec((tb2 * _LPAD, _D), lambda i: (i, 0)),
            vec_spec, vec_spec, vec_spec,
        ],
        out_specs=pl.BlockSpec((tb2, l, _D), lambda i: (i, 0, 0)),
        out_shape=jax.ShapeDtypeStruct((b, l, _D), jnp.float32),
    )(y2, proj_b.reshape(1, _D), ln_gamma.reshape(1, _D),
      ln_beta.reshape(1, _D))
    return out


# speedup vs baseline: 9.6301x; 9.6301x over previous
import functools

import jax
import jax.numpy as jnp
from jax import lax
from jax.experimental import pallas as pl
from jax.experimental.pallas import tpu as pltpu
from jax.experimental.pallas import tpu_sc as plsc

_R, _P, _S, _E = 64, 16, 32, 17
_D = 128
_KP = 144
_OFF_P, _OFF_S, _OFF_E = _R, _R + _P, _R + _P + _S
_LPAD = 56


def _premul_body(tabs_ref, w_ref, out_ref):
    out_ref[...] = jnp.dot(tabs_ref[...], w_ref[...],
                           preferred_element_type=jnp.float32)


def _pack_idx_body(ri_ref, pi_ref, si_ref, ei_ref, out_ref, *, l):
    tb = ri_ref.shape[0]
    pad = jnp.zeros((tb, _D - l), jnp.int32)
    out_ref[:, 0, :] = jnp.concatenate([ri_ref[...], pad], axis=1)
    out_ref[:, 1, :] = jnp.concatenate([pi_ref[...] + _OFF_P, pad], axis=1)
    out_ref[:, 2, :] = jnp.concatenate([si_ref[...] + _OFF_S, pad], axis=1)
    out_ref[:, 3, :] = jnp.concatenate([ei_ref[...] + _OFF_E, pad], axis=1)


def _sc_gather_body(wc_hbm, idx_hbm, y_hbm, wc_sh, idx_s, acc_s, sem_i,
                    sem_g, sem_o, *, rows_per_w, l):
    sid = lax.axis_index("s")
    w = sid * 2 + lax.axis_index("c")
    base = w * rows_per_w

    @pl.when(sid == 0)
    def _():
        pltpu.sync_copy(wc_hbm, wc_sh)
    plsc.subcore_barrier()

    def two_rows(i, carry):
        for s in (0, 1):
            k = i * 2 + s
            b = base + k
            pltpu.async_copy(idx_hbm.at[b], idx_s.at[s], sem_i).wait()
            @pl.when(i >= 1)
            def _():
                pltpu.make_async_copy(
                    y_hbm.at[pl.ds(0, _LPAD), :], acc_s.at[s], sem_o.at[s]).wait()
            pltpu.async_copy(wc_sh.at[idx_s.at[s, 0, pl.ds(0, _LPAD)]],
                             acc_s.at[s], sem_g).wait()
            c1 = pltpu.async_copy(wc_sh.at[idx_s.at[s, 1, pl.ds(0, _LPAD)]],
                                  acc_s.at[s], sem_g, add=True)
            c2 = pltpu.async_copy(wc_sh.at[idx_s.at[s, 2, pl.ds(0, _LPAD)]],
                                  acc_s.at[s], sem_g, add=True)
            c3 = pltpu.async_copy(wc_sh.at[idx_s.at[s, 3, pl.ds(0, _LPAD)]],
                                  acc_s.at[s], sem_g, add=True)
            c1.wait()
            c2.wait()
            c3.wait()
            pltpu.async_copy(acc_s.at[s], y_hbm.at[pl.ds(b * _LPAD, _LPAD), :],
                             sem_o.at[s])
        return carry

    lax.fori_loop(0, rows_per_w // 2, two_rows, 0)
    for s in (0, 1):
        pltpu.make_async_copy(y_hbm.at[pl.ds(0, _LPAD), :], acc_s.at[s],
                              sem_o.at[s]).wait()


def _ln_body(y_ref, b_ref, g_ref, bb_ref, out_ref, *, tb, l):
    x = y_ref[...] + b_ref[0, :][None, :]
    mu = jnp.mean(x, axis=1, keepdims=True)
    xc = x - mu
    var = jnp.mean(xc * xc, axis=1, keepdims=True)
    y = xc * lax.rsqrt(var + 1e-5) * g_ref[0, :][None, :] \
        + bb_ref[0, :][None, :]
    out_ref[...] = y.reshape(tb, _LPAD, _D)[:, :l, :]


def kernel(root_indices, prefix_indices, suffix_indices, ending_indices,
           root_table, prefix_table, suffix_table, ending_table,
           proj_w, proj_b, ln_gamma, ln_beta):
    b, l = root_indices.shape
    tb = 512
    g = b // tb

    tabs = jnp.concatenate([
        root_table, prefix_table, suffix_table, ending_table,
        jnp.zeros((_KP - _OFF_E - _E, _D), jnp.float32),
    ], axis=0)

    wc = pl.pallas_call(
        _premul_body,
        out_shape=jax.ShapeDtypeStruct((_KP, _D), jnp.float32),
    )(tabs, proj_w)

    idx_spec = pl.BlockSpec((tb, l), lambda i: (i, 0))
    idxpack = pl.pallas_call(
        functools.partial(_pack_idx_body, l=l),
        grid=(g,),
        in_specs=[idx_spec, idx_spec, idx_spec, idx_spec],
        out_specs=pl.BlockSpec((tb, 8, _D), lambda i: (i, 0, 0)),
        out_shape=jax.ShapeDtypeStruct((b, 8, _D), jnp.int32),
    )(root_indices, prefix_indices, suffix_indices, ending_indices)

    rows_per_w = b // 32
    mesh = plsc.VectorSubcoreMesh(core_axis_name="c", subcore_axis_name="s")
    y2 = pl.kernel(
        functools.partial(_sc_gather_body, rows_per_w=rows_per_w, l=l),
        out_type=jax.ShapeDtypeStruct((b * _LPAD, _D), jnp.float32),
        mesh=mesh,
        scratch_types=[
            pltpu.VMEM_SHARED((_KP, _D), jnp.float32),
            pltpu.VMEM((2, 8, _D), jnp.int32),
            pltpu.VMEM((2, _LPAD, _D), jnp.float32),
            pltpu.SemaphoreType.DMA,
            pltpu.SemaphoreType.DMA,
            pltpu.SemaphoreType.DMA((2,)),
        ],
    )(wc, idxpack)

    tb2 = 256
    g2 = b // tb2
    vec_spec = pl.BlockSpec((1, _D), lambda i: (0, 0))
    out = pl.pallas_call(
        functools.partial(_ln_body, tb=tb2, l=l),
        grid=(g2,),
        in_specs=[
            pl.BlockSpec((tb2 * _LPAD, _D), lambda i: (i, 0)),
            vec_spec, vec_spec, vec_spec,
        ],
        out_specs=pl.BlockSpec((tb2, l, _D), lambda i: (i, 0, 0)),
        out_shape=jax.ShapeDtypeStruct((b, l, _D), jnp.float32),
    )(y2, proj_b.reshape(1, _D), ln_gamma.reshape(1, _D),
      ln_beta.reshape(1, _D))
    return out


# speedup vs baseline: 10.8350x; 1.1251x over previous
import functools

import jax
import jax.numpy as jnp
from jax import lax
from jax.experimental import pallas as pl
from jax.experimental.pallas import tpu as pltpu
from jax.experimental.pallas import tpu_sc as plsc

_R, _P, _S, _E = 64, 16, 32, 17
_D = 128
_KP = 144
_OFF_P, _OFF_S, _OFF_E = _R, _R + _P, _R + _P + _S
_LPAD = 56


def _premul_body(tabs_ref, w_ref, out_ref):
    out_ref[...] = jnp.dot(tabs_ref[...], w_ref[...],
                           preferred_element_type=jnp.float32)


def _pack_idx_body(ri_ref, pi_ref, si_ref, ei_ref, out_ref, *, l):
    tb = ri_ref.shape[0]
    pad = jnp.zeros((tb, _D - l), jnp.int32)
    out_ref[:, 0, :] = jnp.concatenate([ri_ref[...], pad], axis=1)
    out_ref[:, 1, :] = jnp.concatenate([pi_ref[...] + _OFF_P, pad], axis=1)
    out_ref[:, 2, :] = jnp.concatenate([si_ref[...] + _OFF_S, pad], axis=1)
    out_ref[:, 3, :] = jnp.concatenate([ei_ref[...] + _OFF_E, pad], axis=1)


def _sc_gather_body(wc_hbm, idx_hbm, y_hbm, wc_sh, idx_s, acc_s, sem_i,
                    sem_g, sem_o, *, rows_per_w, l):
    sid = lax.axis_index("s")
    w = sid * 2 + lax.axis_index("c")
    base = w * rows_per_w

    @pl.when(sid == 0)
    def _():
        pltpu.sync_copy(wc_hbm, wc_sh)
    plsc.subcore_barrier()

    def two_chunks(i, carry):
        for s in (0, 1):
            c = i * 2 + s
            b0 = base + c * 4
            pltpu.async_copy(idx_hbm.at[pl.ds(b0, 4)], idx_s.at[s],
                             sem_i).wait()
            @pl.when(i >= 1)
            def _():
                pltpu.make_async_copy(
                    y_hbm.at[pl.ds(0, 4 * _LPAD), :], acc_s.at[s],
                    sem_o.at[s]).wait()
            inits = [
                pltpu.async_copy(
                    wc_sh.at[idx_s.at[s, r, 0, pl.ds(0, _LPAD)]],
                    acc_s.at[s, pl.ds(r * _LPAD, _LPAD)], sem_g)
                for r in range(4)
            ]
            for d in inits:
                d.wait()
            adds = [
                pltpu.async_copy(
                    wc_sh.at[idx_s.at[s, r, f, pl.ds(0, _LPAD)]],
                    acc_s.at[s, pl.ds(r * _LPAD, _LPAD)], sem_g, add=True)
                for f in (1, 2, 3) for r in range(4)
            ]
            for d in adds:
                d.wait()
            pltpu.async_copy(acc_s.at[s], y_hbm.at[pl.ds(b0 * _LPAD, 4 * _LPAD), :],
                             sem_o.at[s])
        return carry

    lax.fori_loop(0, rows_per_w // 8, two_chunks, 0)
    for s in (0, 1):
        pltpu.make_async_copy(y_hbm.at[pl.ds(0, 4 * _LPAD), :], acc_s.at[s],
                              sem_o.at[s]).wait()


def _ln_body(y_ref, b_ref, g_ref, bb_ref, out_ref, *, tb, l):
    x = y_ref[...] + b_ref[0, :][None, :]
    mu = jnp.mean(x, axis=1, keepdims=True)
    xc = x - mu
    var = jnp.mean(xc * xc, axis=1, keepdims=True)
    y = xc * lax.rsqrt(var + 1e-5) * g_ref[0, :][None, :] \
        + bb_ref[0, :][None, :]
    out_ref[...] = y.reshape(tb, _LPAD, _D)[:, :l, :]


def kernel(root_indices, prefix_indices, suffix_indices, ending_indices,
           root_table, prefix_table, suffix_table, ending_table,
           proj_w, proj_b, ln_gamma, ln_beta):
    b, l = root_indices.shape
    tb = 512
    g = b // tb

    tabs = jnp.concatenate([
        root_table, prefix_table, suffix_table, ending_table,
        jnp.zeros((_KP - _OFF_E - _E, _D), jnp.float32),
    ], axis=0)

    wc = pl.pallas_call(
        _premul_body,
        out_shape=jax.ShapeDtypeStruct((_KP, _D), jnp.float32),
    )(tabs, proj_w)

    idx_spec = pl.BlockSpec((tb, l), lambda i: (i, 0))
    idxpack = pl.pallas_call(
        functools.partial(_pack_idx_body, l=l),
        grid=(g,),
        in_specs=[idx_spec, idx_spec, idx_spec, idx_spec],
        out_specs=pl.BlockSpec((tb, 8, _D), lambda i: (i, 0, 0)),
        out_shape=jax.ShapeDtypeStruct((b, 8, _D), jnp.int32),
    )(root_indices, prefix_indices, suffix_indices, ending_indices)

    rows_per_w = b // 32
    mesh = plsc.VectorSubcoreMesh(core_axis_name="c", subcore_axis_name="s")
    y2 = pl.kernel(
        functools.partial(_sc_gather_body, rows_per_w=rows_per_w, l=l),
        out_type=jax.ShapeDtypeStruct((b * _LPAD, _D), jnp.float32),
        mesh=mesh,
        scratch_types=[
            pltpu.VMEM_SHARED((_KP, _D), jnp.float32),
            pltpu.VMEM((2, 4, 8, _D), jnp.int32),
            pltpu.VMEM((2, 4 * _LPAD, _D), jnp.float32),
            pltpu.SemaphoreType.DMA,
            pltpu.SemaphoreType.DMA,
            pltpu.SemaphoreType.DMA((2,)),
        ],
    )(wc, idxpack)

    tb2 = 256
    g2 = b // tb2
    vec_spec = pl.BlockSpec((1, _D), lambda i: (0, 0))
    out = pl.pallas_call(
        functools.partial(_ln_body, tb=tb2, l=l),
        grid=(g2,),
        in_specs=[
            pl.BlockSpec((tb2 * _LPAD, _D), lambda i: (i, 0)),
            vec_spec, vec_spec, vec_spec,
        ],
        out_specs=pl.BlockSpec((tb2, l, _D), lambda i: (i, 0, 0)),
        out_shape=jax.ShapeDtypeStruct((b, l, _D), jnp.float32),
    )(y2, proj_b.reshape(1, _D), ln_gamma.reshape(1, _D),
      ln_beta.reshape(1, _D))
    return out


# speedup vs baseline: 16.3396x; 1.5080x over previous
import functools

import jax
import jax.numpy as jnp
from jax import lax
from jax.experimental import pallas as pl
from jax.experimental.pallas import tpu as pltpu
from jax.experimental.pallas import tpu_sc as plsc

_R, _P, _S, _E = 64, 16, 32, 17
_D = 128
_KP = 144
_OFF_P, _OFF_S, _OFF_E = _R, _R + _P, _R + _P + _S
_LPAD = 56


def _premul_body(tabs_ref, w_ref, out_ref):
    out_ref[...] = jnp.dot(tabs_ref[...], w_ref[...],
                           preferred_element_type=jnp.float32)


def _pack_idx_body(ri_ref, pi_ref, si_ref, ei_ref, out_ref, *, l):
    out_ref[0, :l, :] = ri_ref[...]
    out_ref[1, :l, :] = pi_ref[...] + _OFF_P
    out_ref[2, :l, :] = si_ref[...] + _OFF_S
    out_ref[3, :l, :] = ei_ref[...] + _OFF_E


def _sc_gather_body(wc_hbm, idx_hbm, y_hbm, wc_sh, idx_all, acc_s, sem_i,
                    sem_g, sem_o, *, bsz, l):
    sid = lax.axis_index("s")
    w = sid * 2 + lax.axis_index("c")

    @pl.when(sid == 0)
    def _():
        pltpu.sync_copy(wc_hbm, wc_sh)
    plsc.subcore_barrier()

    pltpu.async_copy(idx_hbm.at[:, :, pl.ds(w * 128, 128)], idx_all,
                     sem_i).wait()

    def two_cols(i, carry):
        for s in (0, 1):
            j = i * 2 + s

            @pl.when(i >= 1)
            def _():
                pltpu.make_async_copy(y_hbm.at[pl.ds(0, 128), :],
                                      acc_s.at[s], sem_o.at[s]).wait()

            pltpu.async_copy(wc_sh.at[idx_all.at[0, j]], acc_s.at[s],
                             sem_g).wait()
            adds = [
                pltpu.async_copy(wc_sh.at[idx_all.at[f, j]], acc_s.at[s],
                                 sem_g, add=True)
                for f in (1, 2, 3)
            ]
            for d in adds:
                d.wait()
            pltpu.async_copy(acc_s.at[s],
                             y_hbm.at[pl.ds(j * bsz + w * 128, 128), :],
                             sem_o.at[s])
        return carry

    lax.fori_loop(0, l // 2, two_cols, 0)
    for s in (0, 1):
        pltpu.make_async_copy(y_hbm.at[pl.ds(0, 128), :], acc_s.at[s],
                              sem_o.at[s]).wait()


def _ln_body(y_ref, b_ref, g_ref, bb_ref, out_ref, *, lb, bsz):
    x = y_ref[...] + b_ref[0, :][None, :]
    mu = jnp.mean(x, axis=1, keepdims=True)
    xc = x - mu
    var = jnp.mean(xc * xc, axis=1, keepdims=True)
    y = xc * lax.rsqrt(var + 1e-5) * g_ref[0, :][None, :] \
        + bb_ref[0, :][None, :]
    out_ref[...] = y.reshape(lb, bsz, _D)


def kernel(root_indices, prefix_indices, suffix_indices, ending_indices,
           root_table, prefix_table, suffix_table, ending_table,
           proj_w, proj_b, ln_gamma, ln_beta):
    b, l = root_indices.shape
    tb = 512
    g = b // tb

    tabs = jnp.concatenate([
        root_table, prefix_table, suffix_table, ending_table,
        jnp.zeros((_KP - _OFF_E - _E, _D), jnp.float32),
    ], axis=0)

    wc = pl.pallas_call(
        _premul_body,
        out_shape=jax.ShapeDtypeStruct((_KP, _D), jnp.float32),
    )(tabs, proj_w)

    idx_spec = pl.BlockSpec((l, tb), lambda i: (0, i))
    idxpack = pl.pallas_call(
        functools.partial(_pack_idx_body, l=l),
        grid=(g,),
        in_specs=[idx_spec, idx_spec, idx_spec, idx_spec],
        out_specs=pl.BlockSpec((4, _LPAD, tb), lambda i: (0, 0, i)),
        out_shape=jax.ShapeDtypeStruct((4, _LPAD, b), jnp.int32),
    )(root_indices.T, prefix_indices.T, suffix_indices.T, ending_indices.T)

    mesh = plsc.VectorSubcoreMesh(core_axis_name="c", subcore_axis_name="s")
    y2 = pl.kernel(
        functools.partial(_sc_gather_body, bsz=b, l=l),
        out_type=jax.ShapeDtypeStruct((l * b, _D), jnp.float32),
        mesh=mesh,
        scratch_types=[
            pltpu.VMEM_SHARED((_KP, _D), jnp.float32),
            pltpu.VMEM((4, _LPAD, 128), jnp.int32),
            pltpu.VMEM((2, 128, _D), jnp.float32),
            pltpu.SemaphoreType.DMA,
            pltpu.SemaphoreType.DMA,
            pltpu.SemaphoreType.DMA((2,)),
        ],
    )(wc, idxpack)

    lb = 2
    g2 = l // lb
    vec_spec = pl.BlockSpec((1, _D), lambda i: (0, 0))
    out_t = pl.pallas_call(
        functools.partial(_ln_body, lb=lb, bsz=b),
        grid=(g2,),
        in_specs=[
            pl.BlockSpec((lb * b, _D), lambda i: (i, 0)),
            vec_spec, vec_spec, vec_spec,
        ],
        out_specs=pl.BlockSpec((lb, b, _D), lambda i: (i, 0, 0)),
        out_shape=jax.ShapeDtypeStruct((l, b, _D), jnp.float32),
    )(y2, proj_b.reshape(1, _D), ln_gamma.reshape(1, _D),
      ln_beta.reshape(1, _D))
    return out_t.transpose(1, 0, 2)
